# baseline (speedup 1.0000x reference)
import jax
import jax.numpy as jnp
from jax.experimental import pallas as pl
from jax.experimental.pallas import tpu as pltpu

_D = 512
_K = 16
_LEVELS = 9
_BT = 512
_CF = 8
_MXU_LEVELS = 6
_HALF_LOG_2PI = 0.9189385332046727

_BASES = [0, 32, 48, 56, 60, 62]
_NCHUNKS = 63


def _softmax_lanes(w):
    wmax = jnp.max(w, axis=-1, keepdims=True)
    we = jnp.exp(w - wmax)
    return we / jnp.sum(we, axis=-1, keepdims=True)


def _circuit_body(xt_ref, mu_ref, a_ref, bc_ref, *rest):
    w_refs = rest[:_LEVELS]
    out_ref = rest[_LEVELS]
    s_a, s_b, m_a, m_b, bd_ref = rest[_LEVELS + 1 :]

    @pl.when(pl.program_id(0) == 0)
    def _build():
        row_f = jax.lax.broadcasted_iota(jnp.int32, (_CF, _K, _CF * _K), 0)
        col_f = jax.lax.broadcasted_iota(jnp.int32, (_CF, _K, _CF * _K), 2) // _K
        keep = row_f == col_f
        for l in range(_MXU_LEVELS):
            base = _BASES[l]

            def build_chunk(c, carry, w_ref=w_refs[l], base=base):
                sw = _softmax_lanes(w_ref[pl.ds(c * _CF, _CF)])
                tiled = jnp.concatenate([sw] * _CF, axis=2)
                bd = jnp.where(keep, tiled, 0.0).reshape(_CF * _K, _CF * _K)
                bd_ref[pl.ds(base + c, 1)] = bd[None]
                return carry

            jax.lax.fori_loop(0, (_D >> (l + 1)) // _CF, build_chunk, 0)

    cin = 2 * _CF

    def in_chunk(i, carry):
        f0 = i * cin
        xv = xt_ref[pl.ds(f0, cin), :]
        mu = mu_ref[pl.ds(f0, cin)]
        av = a_ref[pl.ds(f0, cin)]
        bc = bc_ref[pl.ds(f0, cin)]
        diff = xv[:, None, :] - mu
        s_a[pl.ds(f0, cin)] = jnp.exp(av * (diff * diff) - bc)
        return carry

    jax.lax.fori_loop(0, _D // cin, in_chunk, 0)

    src_s, src_m, dst_s, dst_m = s_a, m_a, s_b, m_b
    for l0, l1 in ((0, 1), (2, 3), (4, 5)):
        base0 = _BASES[l0]
        base1 = _BASES[l1]
        renorm = l0 in (0, 4)
        first = l0 == 0
        dot_dims = (((1,), (0,)), ((), ()))

        def fused(c, carry, base0=base0, base1=base1, renorm=renorm,
                  first=first, src_s=src_s, src_m=src_m,
                  dst_s=dst_s, dst_m=dst_m):
            f0 = c * 4 * _CF
            sp = src_s[pl.ds(f0, 4 * _CF)].reshape(2 * _CF, 2, _K, _BT)
            t = sp[:, 0] * sp[:, 1]
            if renorm:
                tmax = jnp.max(t, axis=1, keepdims=True)
                p = (t * (1.0 / tmax)).reshape(2 * _CF * _K, _BT)
                tm = tmax.reshape(_CF, 2, 1, _BT)
                logterm = jnp.log(tm[:, 0] * tm[:, 1])
            else:
                p = t.reshape(2 * _CF * _K, _BT)
            bd0 = bd_ref[pl.ds(base0 + 2 * c, 1)][0]
            bd1 = bd_ref[pl.ds(base0 + 2 * c + 1, 1)][0]
            s20 = jax.lax.dot_general(
                bd0, p[: _CF * _K], dot_dims,
                preferred_element_type=jnp.float32)
            s21 = jax.lax.dot_general(
                bd1, p[_CF * _K :], dot_dims,
                preferred_element_type=jnp.float32)
            sa = s20.reshape(_CF // 2, 2, _K, _BT)
            sb = s21.reshape(_CF // 2, 2, _K, _BT)
            t2 = jnp.concatenate([sa[:, 0] * sa[:, 1],
                                  sb[:, 0] * sb[:, 1]], axis=0)
            p2 = t2.reshape(_CF * _K, _BT)
            bd2 = bd_ref[pl.ds(base1 + c, 1)][0]
            s22 = jax.lax.dot_general(
                bd2, p2, dot_dims, preferred_element_type=jnp.float32)
            dst_s[pl.ds(c * _CF, _CF)] = s22.reshape(_CF, _K, _BT)
            if first:
                dst_m[pl.ds(c * _CF, _CF)] = logterm
            else:
                mp = src_m[pl.ds(f0, 4 * _CF)].reshape(_CF, 4, 1, _BT)
                msum = (mp[:, 0] + mp[:, 1]) + (mp[:, 2] + mp[:, 3])
                if renorm:
                    msum = msum + logterm
                dst_m[pl.ds(c * _CF, _CF)] = msum
            return carry

        jax.lax.fori_loop(0, (_D >> (l1 + 1)) // _CF, fused, 0)
        src_s, src_m, dst_s, dst_m = dst_s, dst_m, src_s, src_m

    for l in range(_MXU_LEVELS, _LEVELS):
        f2 = _D >> (l + 1)
        sw = _softmax_lanes(w_refs[l][...])
        sp = src_s[0 : 2 * f2].reshape(f2, 2, _K, _BT)
        t = sp[:, 0] * sp[:, 1]
        mp = src_m[0 : 2 * f2].reshape(f2, 2, 1, _BT)
        dst_m[0:f2] = mp[:, 0] + mp[:, 1]
        acc = sw[:, :, 0:1] * t[:, 0:1, :]
        for k in range(1, _K):
            acc = acc + sw[:, :, k : k + 1] * t[:, k : k + 1, :]
        dst_s[0:f2] = acc
        src_s, src_m, dst_s, dst_m = dst_s, dst_m, src_s, src_m

    out_ref[...] = (jnp.log(src_s[0:1]) + src_m[0:1])[0]


def kernel(x, mu, log_sigma, W0, W1, W2, W3, W4, W5, W6, W7, W8):
    b, c, d = x.shape
    ws = [W0, W1, W2, W3, W4, W5, W6, W7, W8]
    xt = jnp.transpose(x[:, 0, :])
    mu3 = mu[:, :, None]
    a3 = (-0.5 * jnp.exp(-2.0 * log_sigma))[:, :, None]
    bc3 = (log_sigma + _HALF_LOG_2PI)[:, :, None]

    grid = (b // _BT,)

    in_specs = [
        pl.BlockSpec((d, _BT), lambda i: (0, i)),
        pl.BlockSpec((d, _K, 1), lambda i: (0, 0, 0)),
        pl.BlockSpec((d, _K, 1), lambda i: (0, 0, 0)),
        pl.BlockSpec((d, _K, 1), lambda i: (0, 0, 0)),
    ]
    for w in ws:
        in_specs.append(pl.BlockSpec(w.shape, lambda i: (0, 0, 0)))

    scratch_shapes = [
        pltpu.VMEM((d, _K, _BT), jnp.float32),
        pltpu.VMEM((d // 2, _K, _BT), jnp.float32),
        pltpu.VMEM((d, 1, _BT), jnp.float32),
        pltpu.VMEM((d // 2, 1, _BT), jnp.float32),
        pltpu.VMEM((_NCHUNKS, _CF * _K, _CF * _K), jnp.float32),
    ]

    out = pl.pallas_call(
        _circuit_body,
        grid=grid,
        in_specs=in_specs,
        out_specs=pl.BlockSpec((_K, _BT), lambda i: (0, i)),
        out_shape=jax.ShapeDtypeStruct((_K, b), jnp.float32),
        scratch_shapes=scratch_shapes,
    )(xt, mu3, a3, bc3, *ws)

    return jnp.transpose(out).reshape(b, c, _K)


# speedup vs baseline: 1.0055x; 1.0055x over previous
import jax
import jax.numpy as jnp
from jax.experimental import pallas as pl
from jax.experimental.pallas import tpu as pltpu

_D = 512
_K = 16
_LEVELS = 9
_BT = 512
_CF = 8
_MXU_LEVELS = 6
_HALF_LOG_2PI = 0.9189385332046727

_BASES = [0, 32, 48, 56, 60, 62]
_NCHUNKS = 63


def _softmax_lanes(w):
    wmax = jnp.max(w, axis=-1, keepdims=True)
    we = jnp.exp(w - wmax)
    return we / jnp.sum(we, axis=-1, keepdims=True)


def _circuit_body(xt_ref, mu_ref, a_ref, bc_ref, *rest):
    w_refs = rest[:_LEVELS]
    out_ref = rest[_LEVELS]
    s_a, s_b, m_a, m_b, bd_ref = rest[_LEVELS + 1 :]

    @pl.when(pl.program_id(0) == 0)
    def _build():
        row_f = jax.lax.broadcasted_iota(jnp.int32, (_CF, _K, _CF * _K), 0)
        col_f = jax.lax.broadcasted_iota(jnp.int32, (_CF, _K, _CF * _K), 2) // _K
        keep = row_f == col_f
        for l in range(_MXU_LEVELS):
            base = _BASES[l]

            def build_chunk(c, carry, w_ref=w_refs[l], base=base):
                sw = _softmax_lanes(w_ref[pl.ds(c * _CF, _CF)])
                tiled = jnp.concatenate([sw] * _CF, axis=2)
                bd = jnp.where(keep, tiled, 0.0).reshape(_CF * _K, _CF * _K)
                bd_ref[pl.ds(base + c, 1)] = bd[None]
                return carry

            jax.lax.fori_loop(0, (_D >> (l + 1)) // _CF, build_chunk, 0)

    cin = 2 * _CF

    def in_chunk(i, carry):
        f0 = i * cin
        xv = xt_ref[pl.ds(f0, cin), :]
        mu = mu_ref[pl.ds(f0, cin)]
        av = a_ref[pl.ds(f0, cin)]
        bc = bc_ref[pl.ds(f0, cin)]
        diff = xv[:, None, :] - mu
        s_a[pl.ds(f0, cin)] = jnp.exp(av * (diff * diff) - bc)
        return carry

    jax.lax.fori_loop(0, _D // cin, in_chunk, 0)

    src_s, src_m, dst_s, dst_m = s_a, m_a, s_b, m_b
    for l in range(_MXU_LEVELS):
        base = _BASES[l]
        f2 = _D >> (l + 1)
        npair = f2 // (2 * _CF)
        renorm = l in (0, 2, 4)
        first = l == 0

        def mxu_pair(c, carry, base=base, renorm=renorm, first=first,
                     src_s=src_s, src_m=src_m, dst_s=dst_s, dst_m=dst_m):
            f0 = c * 2 * _CF
            sp = src_s[pl.ds(2 * f0, 4 * _CF)].reshape(2 * _CF, 2, _K, _BT)
            t = sp[:, 0] * sp[:, 1]
            if first:
                msum = 0.0
            else:
                mp = src_m[pl.ds(2 * f0, 4 * _CF)].reshape(2 * _CF, 2, 1, _BT)
                msum = mp[:, 0] + mp[:, 1]
            if renorm:
                tmax = jnp.max(t, axis=1, keepdims=True)
                dst_m[pl.ds(f0, 2 * _CF)] = msum + jnp.log(tmax)
                p = (t * (1.0 / tmax)).reshape(2 * _CF * _K, _BT)
            else:
                dst_m[pl.ds(f0, 2 * _CF)] = msum
                p = t.reshape(2 * _CF * _K, _BT)
            bd0 = bd_ref[pl.ds(base + 2 * c, 1)][0]
            bd1 = bd_ref[pl.ds(base + 2 * c + 1, 1)][0]
            s20 = jax.lax.dot_general(
                bd0, p[: _CF * _K], (((1,), (0,)), ((), ())),
                preferred_element_type=jnp.float32,
            )
            s21 = jax.lax.dot_general(
                bd1, p[_CF * _K :], (((1,), (0,)), ((), ())),
                preferred_element_type=jnp.float32,
            )
            dst_s[pl.ds(f0, _CF)] = s20.reshape(_CF, _K, _BT)
            dst_s[pl.ds(f0 + _CF, _CF)] = s21.reshape(_CF, _K, _BT)
            return carry

        def mxu_single(c, carry, base=base, renorm=renorm,
                       src_s=src_s, src_m=src_m, dst_s=dst_s, dst_m=dst_m):
            f0 = c * _CF
            sp = src_s[pl.ds(2 * f0, 2 * _CF)].reshape(_CF, 2, _K, _BT)
            t = sp[:, 0] * sp[:, 1]
            mp = src_m[pl.ds(2 * f0, 2 * _CF)].reshape(_CF, 2, 1, _BT)
            msum = mp[:, 0] + mp[:, 1]
            if renorm:
                tmax = jnp.max(t, axis=1, keepdims=True)
                dst_m[pl.ds(f0, _CF)] = msum + jnp.log(tmax)
                p = (t * (1.0 / tmax)).reshape(_CF * _K, _BT)
            else:
                dst_m[pl.ds(f0, _CF)] = msum
                p = t.reshape(_CF * _K, _BT)
            bd = bd_ref[pl.ds(base + c, 1)][0]
            s2 = jax.lax.dot_general(
                bd, p, (((1,), (0,)), ((), ())),
                preferred_element_type=jnp.float32,
            )
            dst_s[pl.ds(f0, _CF)] = s2.reshape(_CF, _K, _BT)
            return carry

        if npair >= 1:
            jax.lax.fori_loop(0, npair, mxu_pair, 0)
        else:
            jax.lax.fori_loop(0, f2 // _CF, mxu_single, 0)
        src_s, src_m, dst_s, dst_m = dst_s, dst_m, src_s, src_m

    for l in range(_MXU_LEVELS, _LEVELS):
        f2 = _D >> (l + 1)
        sw = _softmax_lanes(w_refs[l][...])
        sp = src_s[0 : 2 * f2].reshape(f2, 2, _K, _BT)
        t = sp[:, 0] * sp[:, 1]
        mp = src_m[0 : 2 * f2].reshape(f2, 2, 1, _BT)
        dst_m[0:f2] = mp[:, 0] + mp[:, 1]
        acc = sw[:, :, 0:1] * t[:, 0:1, :]
        for k in range(1, _K):
            acc = acc + sw[:, :, k : k + 1] * t[:, k : k + 1, :]
        dst_s[0:f2] = acc
        src_s, src_m, dst_s, dst_m = dst_s, dst_m, src_s, src_m

    out_ref[...] = (jnp.log(src_s[0:1]) + src_m[0:1])[0]


def kernel(x, mu, log_sigma, W0, W1, W2, W3, W4, W5, W6, W7, W8):
    b, c, d = x.shape
    ws = [W0, W1, W2, W3, W4, W5, W6, W7, W8]
    xt = jnp.transpose(x[:, 0, :])
    mu3 = mu[:, :, None]
    a3 = (-0.5 * jnp.exp(-2.0 * log_sigma))[:, :, None]
    bc3 = (log_sigma + _HALF_LOG_2PI)[:, :, None]

    grid = (b // _BT,)

    in_specs = [
        pl.BlockSpec((d, _BT), lambda i: (0, i)),
        pl.BlockSpec((d, _K, 1), lambda i: (0, 0, 0)),
        pl.BlockSpec((d, _K, 1), lambda i: (0, 0, 0)),
        pl.BlockSpec((d, _K, 1), lambda i: (0, 0, 0)),
    ]
    for w in ws:
        in_specs.append(pl.BlockSpec(w.shape, lambda i: (0, 0, 0)))

    scratch_shapes = [
        pltpu.VMEM((d, _K, _BT), jnp.float32),
        pltpu.VMEM((d // 2, _K, _BT), jnp.float32),
        pltpu.VMEM((d, 1, _BT), jnp.float32),
        pltpu.VMEM((d // 2, 1, _BT), jnp.float32),
        pltpu.VMEM((_NCHUNKS, _CF * _K, _CF * _K), jnp.float32),
    ]

    out = pl.pallas_call(
        _circuit_body,
        grid=grid,
        in_specs=in_specs,
        out_specs=pl.BlockSpec((_K, _BT), lambda i: (0, i)),
        out_shape=jax.ShapeDtypeStruct((_K, b), jnp.float32),
        scratch_shapes=scratch_shapes,
    )(xt, mu3, a3, bc3, *ws)

    return jnp.transpose(out).reshape(b, c, _K)
